# baseline (speedup 1.0000x reference)
import functools

import jax
import jax.numpy as jnp
from jax import lax
from jax.experimental import pallas as pl
from jax.experimental.pallas import tpu as pltpu
from jax.experimental.pallas import tpu_sc as plsc

H = 2048
E = 16
K = 2
CHUNK = 2048
T = 16384
NW = 32
TPT = T // NW
NG = TPT // 16


def _dense_body(x_ref, w_ref, pt_ref):
    logits = jnp.dot(x_ref[...], w_ref[...],
                     preferred_element_type=jnp.float32)
    lt = logits.T

    m = jnp.max(lt, axis=0, keepdims=True)
    e = jnp.exp(lt - m)
    s = jnp.sum(e, axis=0, keepdims=True)
    pt_ref[...] = e / s


def _dense_stage(x, W):
    return pl.pallas_call(
        _dense_body,
        grid=(T // CHUNK,),
        in_specs=[
            pl.BlockSpec((CHUNK, H), lambda i: (i, 0)),
            pl.BlockSpec((H, E), lambda i: (0, 0)),
        ],
        out_specs=pl.BlockSpec((E, CHUNK), lambda i: (0, i)),
        out_shape=jax.ShapeDtypeStruct((E, T), jnp.float32),
    )(x, W)


def _route_body(pt_hbm, w1_hbm, w2_hbm, i1_hbm, i2_hbm, acc_hbm,
                slab, w1v, w2v, i1v, i2v, acc2d, sem):
    wid = lax.axis_index("s") * 2 + lax.axis_index("c")
    base = wid * TPT
    pltpu.async_copy(pt_hbm.at[:, pl.ds(base, TPT)], slab, sem).wait()

    zeros = jnp.zeros((16,), jnp.float32)

    def group(g, carry):
        cnt, ps = carry
        tb = g * 16
        pv = [slab[e, pl.ds(tb, 16)] for e in range(E)]
        b = pv[0]
        bi = jnp.zeros((16,), jnp.int32)
        for e in range(1, E):
            c = pv[e] > b
            b = jnp.where(c, pv[e], b)
            bi = jnp.where(c, e, bi)
        s2 = jnp.full((16,), -jnp.inf, jnp.float32)
        si = jnp.zeros((16,), jnp.int32)
        for e in range(E):
            c = (pv[e] > s2) & (bi != e)
            s2 = jnp.where(c, pv[e], s2)
            si = jnp.where(c, e, si)
        tot = b + s2
        w1v[pl.ds(tb, 16)] = b / tot
        w2v[pl.ds(tb, 16)] = s2 / tot
        i1v[pl.ds(tb, 16)] = bi
        i2v[pl.ds(tb, 16)] = si
        one = jnp.float32(1.0)
        cnt = tuple(
            cnt[e]
            + jnp.where(bi == e, one, 0.0)
            + jnp.where(si == e, one, 0.0)
            for e in range(E))
        ps = tuple(ps[e] + pv[e] for e in range(E))
        return cnt, ps

    cnt, ps = lax.fori_loop(0, NG, group,
                            (tuple([zeros] * E), tuple([zeros] * E)))
    for e in range(E):
        acc2d[e, :] = cnt[e]
        acc2d[E + e, :] = ps[e]

    pltpu.sync_copy(w1v, w1_hbm.at[pl.ds(base, TPT)])
    pltpu.sync_copy(w2v, w2_hbm.at[pl.ds(base, TPT)])
    pltpu.sync_copy(i1v, i1_hbm.at[pl.ds(base, TPT)])
    pltpu.sync_copy(i2v, i2_hbm.at[pl.ds(base, TPT)])
    pltpu.sync_copy(acc2d, acc_hbm.at[wid])


_route_stage = functools.partial(
    pl.kernel,
    _route_body,
    out_type=[
        jax.ShapeDtypeStruct((T,), jnp.float32),
        jax.ShapeDtypeStruct((T,), jnp.float32),
        jax.ShapeDtypeStruct((T,), jnp.int32),
        jax.ShapeDtypeStruct((T,), jnp.int32),
        jax.ShapeDtypeStruct((NW, 2 * E, 16), jnp.float32),
    ],
    mesh=plsc.VectorSubcoreMesh(core_axis_name="c", subcore_axis_name="s"),
    scratch_types=[
        pltpu.VMEM((E, TPT), jnp.float32),
        pltpu.VMEM((TPT,), jnp.float32),
        pltpu.VMEM((TPT,), jnp.float32),
        pltpu.VMEM((TPT,), jnp.int32),
        pltpu.VMEM((TPT,), jnp.int32),
        pltpu.VMEM((2 * E, 16), jnp.float32),
        pltpu.SemaphoreType.DMA,
    ],
)()


def kernel(hidden_states, W):
    B, S, _ = hidden_states.shape
    x = hidden_states.reshape(T, H)
    pt = _dense_stage(x, W)
    w1, w2, i1, i2, acc = _route_stage(pt)
    top_k_weights = jnp.stack([w1, w2], axis=-1).reshape(B, S, K)
    top_k_indices = jnp.stack([i1, i2], axis=-1).reshape(B, S, K)
    cnt_tot = jnp.sum(acc[:, :E, :], axis=(0, 2))
    ps_tot = jnp.sum(acc[:, E:, :], axis=(0, 2))
    aux = jnp.sum(cnt_tot * ps_tot) * E / (T * T)
    return top_k_weights, top_k_indices, aux


# speedup vs baseline: 1.0150x; 1.0150x over previous
import functools

import jax
import jax.numpy as jnp
from jax import lax
from jax.experimental import pallas as pl
from jax.experimental.pallas import tpu as pltpu
from jax.experimental.pallas import tpu_sc as plsc

H = 2048
E = 16
K = 2
CHUNK = 1024
T = 16384
NW = 32
TPT = T // NW
NG = TPT // 16


def _dense_body(x_ref, w_ref, pt_ref):
    logits = jnp.dot(x_ref[...], w_ref[...],
                     preferred_element_type=jnp.float32)
    lt = logits.T

    m = jnp.max(lt, axis=0, keepdims=True)
    e = jnp.exp(lt - m)
    s = jnp.sum(e, axis=0, keepdims=True)
    pt_ref[...] = e / s


def _dense_stage(x, W):
    return pl.pallas_call(
        _dense_body,
        grid=(T // CHUNK,),
        in_specs=[
            pl.BlockSpec((CHUNK, H), lambda i: (i, 0)),
            pl.BlockSpec((H, E), lambda i: (0, 0)),
        ],
        out_specs=pl.BlockSpec((E, CHUNK), lambda i: (0, i)),
        out_shape=jax.ShapeDtypeStruct((E, T), jnp.float32),
    )(x, W)


def _route_body(pt_hbm, w1_hbm, w2_hbm, i1_hbm, i2_hbm, acc_hbm,
                slab, w1v, w2v, i1v, i2v, acc2d, sem):
    wid = lax.axis_index("s") * 2 + lax.axis_index("c")
    base = wid * TPT
    pltpu.async_copy(pt_hbm.at[:, pl.ds(base, TPT)], slab, sem).wait()

    zeros = jnp.zeros((16,), jnp.float32)

    def group(g, carry):
        cnt, ps = carry
        tb = g * 16
        pv = [slab[e, pl.ds(tb, 16)] for e in range(E)]
        b = pv[0]
        bi = jnp.zeros((16,), jnp.int32)
        for e in range(1, E):
            c = pv[e] > b
            b = jnp.where(c, pv[e], b)
            bi = jnp.where(c, e, bi)
        s2 = jnp.full((16,), -jnp.inf, jnp.float32)
        si = jnp.zeros((16,), jnp.int32)
        for e in range(E):
            c = (pv[e] > s2) & (bi != e)
            s2 = jnp.where(c, pv[e], s2)
            si = jnp.where(c, e, si)
        tot = b + s2
        w1v[pl.ds(tb, 16)] = b / tot
        w2v[pl.ds(tb, 16)] = s2 / tot
        i1v[pl.ds(tb, 16)] = bi
        i2v[pl.ds(tb, 16)] = si
        one = jnp.float32(1.0)
        cnt = tuple(
            cnt[e]
            + jnp.where(bi == e, one, 0.0)
            + jnp.where(si == e, one, 0.0)
            for e in range(E))
        ps = tuple(ps[e] + pv[e] for e in range(E))
        return cnt, ps

    cnt, ps = lax.fori_loop(0, NG, group,
                            (tuple([zeros] * E), tuple([zeros] * E)))
    for e in range(E):
        acc2d[e, :] = cnt[e]
        acc2d[E + e, :] = ps[e]

    pltpu.sync_copy(w1v, w1_hbm.at[pl.ds(base, TPT)])
    pltpu.sync_copy(w2v, w2_hbm.at[pl.ds(base, TPT)])
    pltpu.sync_copy(i1v, i1_hbm.at[pl.ds(base, TPT)])
    pltpu.sync_copy(i2v, i2_hbm.at[pl.ds(base, TPT)])
    pltpu.sync_copy(acc2d, acc_hbm.at[wid])


_route_stage = functools.partial(
    pl.kernel,
    _route_body,
    out_type=[
        jax.ShapeDtypeStruct((T,), jnp.float32),
        jax.ShapeDtypeStruct((T,), jnp.float32),
        jax.ShapeDtypeStruct((T,), jnp.int32),
        jax.ShapeDtypeStruct((T,), jnp.int32),
        jax.ShapeDtypeStruct((NW, 2 * E, 16), jnp.float32),
    ],
    mesh=plsc.VectorSubcoreMesh(core_axis_name="c", subcore_axis_name="s"),
    scratch_types=[
        pltpu.VMEM((E, TPT), jnp.float32),
        pltpu.VMEM((TPT,), jnp.float32),
        pltpu.VMEM((TPT,), jnp.float32),
        pltpu.VMEM((TPT,), jnp.int32),
        pltpu.VMEM((TPT,), jnp.int32),
        pltpu.VMEM((2 * E, 16), jnp.float32),
        pltpu.SemaphoreType.DMA,
    ],
)()


def kernel(hidden_states, W):
    B, S, _ = hidden_states.shape
    x = hidden_states.reshape(T, H)
    pt = _dense_stage(x, W)
    w1, w2, i1, i2, acc = _route_stage(pt)
    top_k_weights = jnp.stack([w1, w2], axis=-1).reshape(B, S, K)
    top_k_indices = jnp.stack([i1, i2], axis=-1).reshape(B, S, K)
    cnt_tot = jnp.sum(acc[:, :E, :], axis=(0, 2))
    ps_tot = jnp.sum(acc[:, E:, :], axis=(0, 2))
    aux = jnp.sum(cnt_tot * ps_tot) * E / (T * T)
    return top_k_weights, top_k_indices, aux
